# Initial kernel scaffold; baseline (speedup 1.0000x reference)
import jax
import jax.numpy as jnp
from jax.experimental import pallas as pl


def kernel(x, gate_w, w1, w2, w3, expert_bias):
    raise NotImplementedError("write your pallas kernel here")


# speedup vs baseline: 3.9416x; 3.9416x over previous
import functools

import jax
import jax.numpy as jnp
from jax import lax
from jax.experimental import pallas as pl
from jax.experimental.pallas import tpu as pltpu
from jax.experimental.pallas import tpu_sc as plsc

E = 8
K = 2
DIM = 1024
HID = 2816
N_TOK = 8192
N_SLOT = N_TOK * K

ROW_BLK = 256
NB = N_SLOT // ROW_BLK
T_MAX = NB + E - 1

NW = 32
GCH = 64
CCH = 32


def _router_body(x_ref, gw_ref, bias_ref, sel_ref, sc_ref):
    logits = jnp.dot(x_ref[...].astype(jnp.bfloat16),
                     gw_ref[...].astype(jnp.bfloat16),
                     preferred_element_type=jnp.float32)
    m = jnp.max(logits, axis=1, keepdims=True)
    ex = jnp.exp(logits - m)
    scores = ex / jnp.sum(ex, axis=1, keepdims=True)
    biased = scores + bias_ref[...]
    cols = lax.broadcasted_iota(jnp.int32, biased.shape, 1)
    i1 = jnp.argmax(biased, axis=1).astype(jnp.int32)
    m1 = cols == i1[:, None]
    b2 = jnp.where(m1, -jnp.inf, biased)
    i2 = jnp.argmax(b2, axis=1).astype(jnp.int32)
    s1 = jnp.sum(jnp.where(m1, scores, 0.0), axis=1)
    s2 = jnp.sum(jnp.where(cols == i2[:, None], scores, 0.0), axis=1)
    sel_ref[...] = jnp.stack([i1, i2])
    sc_ref[...] = jnp.stack([s1, s2])


def _router(xt, gw_t, bias2d):
    TB = 1024
    return pl.pallas_call(
        _router_body,
        grid=(N_TOK // TB,),
        in_specs=[
            pl.BlockSpec((TB, DIM), lambda i: (i, 0)),
            pl.BlockSpec((DIM, E), lambda i: (0, 0)),
            pl.BlockSpec((1, E), lambda i: (0, 0)),
        ],
        out_specs=[
            pl.BlockSpec((K, TB), lambda i: (0, i)),
            pl.BlockSpec((K, TB), lambda i: (0, i)),
        ],
        out_shape=[
            jax.ShapeDtypeStruct((K, N_TOK), jnp.int32),
            jax.ShapeDtypeStruct((K, N_TOK), jnp.float32),
        ],
    )(xt, gw_t, bias2d)


def _sc_gather_body(xt_hbm, idx_hbm, out_hbm, idx_v, rows_v, sem):
    wid = lax.axis_index("s") * 2 + lax.axis_index("c")
    base = wid * (N_SLOT // NW)
    for c in range(N_SLOT // NW // GCH):
        o = base + c * GCH
        pltpu.sync_copy(idx_hbm.at[pl.ds(o, GCH)], idx_v)
        pltpu.async_copy(xt_hbm.at[idx_v], rows_v, sem).wait()
        pltpu.sync_copy(rows_v, out_hbm.at[pl.ds(o, GCH)])


def _sc_gather(xt, src_token):
    return functools.partial(
        pl.kernel,
        out_type=jax.ShapeDtypeStruct((N_SLOT, DIM), jnp.float32),
        mesh=plsc.VectorSubcoreMesh(core_axis_name="c", subcore_axis_name="s"),
        scratch_types=[
            pltpu.VMEM((GCH,), jnp.int32),
            pltpu.VMEM((GCH, DIM), jnp.float32),
            pltpu.SemaphoreType.DMA,
        ],
    )(_sc_gather_body)(xt, src_token)


def _gmm_body(meta_ref, xs_ref, s_ref, w1_ref, w3_ref, w2_ref, out_ref):
    t = pl.program_id(0)
    lo = meta_ref[2, t]
    hi = meta_ref[3, t]
    first = meta_ref[4, t]

    @pl.when(first == 1)
    def _():
        out_ref[...] = jnp.zeros_like(out_ref)

    @pl.when(hi > lo)
    def _():
        xb = (xs_ref[...] * s_ref[...]).astype(jnp.bfloat16)
        a = jnp.dot(xb, w1_ref[0], preferred_element_type=jnp.float32)
        b = jnp.dot(xb, w3_ref[0], preferred_element_type=jnp.float32)
        h = (a * jax.nn.sigmoid(a) * b).astype(jnp.bfloat16)
        o = jnp.dot(h, w2_ref[0], preferred_element_type=jnp.float32)
        rows = lax.broadcasted_iota(jnp.int32, (ROW_BLK, DIM), 0)
        mask = (rows >= lo) & (rows < hi)
        out_ref[...] = out_ref[...] + jnp.where(mask, o, 0.0)


def _gmm(meta, xs, s_sorted, w1b, w3b, w2b):
    grid_spec = pltpu.PrefetchScalarGridSpec(
        num_scalar_prefetch=1,
        grid=(T_MAX,),
        in_specs=[
            pl.BlockSpec((ROW_BLK, DIM), lambda t, m: (m[1, t], 0)),
            pl.BlockSpec((ROW_BLK, 1), lambda t, m: (m[1, t], 0)),
            pl.BlockSpec((1, DIM, HID), lambda t, m: (m[0, t], 0, 0)),
            pl.BlockSpec((1, DIM, HID), lambda t, m: (m[0, t], 0, 0)),
            pl.BlockSpec((1, HID, DIM), lambda t, m: (m[0, t], 0, 0)),
        ],
        out_specs=pl.BlockSpec((ROW_BLK, DIM), lambda t, m: (m[1, t], 0)),
    )
    return pl.pallas_call(
        _gmm_body,
        grid_spec=grid_spec,
        out_shape=jax.ShapeDtypeStruct((N_SLOT, DIM), jnp.float32),
    )(meta, xs, s_sorted, w1b, w3b, w2b)


def _sc_combine_body(ro_hbm, d0_hbm, d1_hbm, out_hbm, i0_v, i1_v, r0_v, r1_v, sem):
    wid = lax.axis_index("s") * 2 + lax.axis_index("c")
    base = wid * (N_TOK // NW)
    for c in range(N_TOK // NW // CCH):
        o = base + c * CCH
        pltpu.sync_copy(d0_hbm.at[pl.ds(o, CCH)], i0_v)
        pltpu.sync_copy(d1_hbm.at[pl.ds(o, CCH)], i1_v)
        pltpu.async_copy(ro_hbm.at[i0_v], r0_v, sem).wait()
        pltpu.async_copy(ro_hbm.at[i1_v], r1_v, sem).wait()

        def row_add(i, _):
            for j in range(DIM // 16):
                sl = pl.ds(j * 16, 16)
                r0_v[i, sl] = r0_v[i, sl] + r1_v[i, sl]
            return 0

        lax.fori_loop(0, CCH, row_add, 0)
        pltpu.sync_copy(r0_v, out_hbm.at[pl.ds(o, CCH)])


def _sc_combine(ro, d0, d1):
    return functools.partial(
        pl.kernel,
        out_type=jax.ShapeDtypeStruct((N_TOK, DIM), jnp.float32),
        mesh=plsc.VectorSubcoreMesh(core_axis_name="c", subcore_axis_name="s"),
        scratch_types=[
            pltpu.VMEM((CCH,), jnp.int32),
            pltpu.VMEM((CCH,), jnp.int32),
            pltpu.VMEM((CCH, DIM), jnp.float32),
            pltpu.VMEM((CCH, DIM), jnp.float32),
            pltpu.SemaphoreType.DMA,
        ],
    )(_sc_combine_body)(ro, d0, d1)


def _tile_metadata(counts):
    counts = counts.astype(jnp.int32)
    off = jnp.concatenate([jnp.zeros((1,), jnp.int32), jnp.cumsum(counts)])
    start_blk = off[:E] // ROW_BLK
    end_blk = (off[1:] + ROW_BLK - 1) // ROW_BLK
    nblk = jnp.where(counts > 0, end_blk - start_blk, 0)
    ts = jnp.concatenate([jnp.zeros((1,), jnp.int32),
                          jnp.cumsum(nblk)[:-1].astype(jnp.int32)])
    total = ts[E - 1] + nblk[E - 1]
    t = jnp.arange(T_MAX, dtype=jnp.int32)
    e_t = jnp.clip(jnp.searchsorted(ts, t, side="right").astype(jnp.int32) - 1, 0, E - 1)
    blk = start_blk[e_t] + (t - ts[e_t])
    active = t < total
    lo = jnp.maximum(off[e_t], blk * ROW_BLK) - blk * ROW_BLK
    hi = jnp.minimum(off[e_t + 1], (blk + 1) * ROW_BLK) - blk * ROW_BLK
    lo = jnp.where(active, lo, 0)
    hi = jnp.where(active, hi, 0)
    first = (active & (off[e_t] <= blk * ROW_BLK)).astype(jnp.int32)
    blk = jnp.where(active, blk, NB - 1)
    return jnp.stack([e_t, blk, lo, hi, first]).astype(jnp.int32)


def kernel(x, gate_w, w1, w2, w3, expert_bias):
    bs, slen, dim = x.shape
    xt = x.reshape(N_TOK, DIM)

    sel, scores = _router(xt, gate_w.T, expert_bias.reshape(1, E))

    slot_expert = jnp.concatenate([sel[0], sel[1]])
    sorted_src = jnp.argsort(slot_expert, stable=True).astype(jnp.int32)
    src_token = jnp.bitwise_and(sorted_src, N_TOK - 1)
    s_flat = jnp.concatenate([scores[0], scores[1]])
    s_sorted = s_flat[sorted_src].reshape(N_SLOT, 1)
    dest = jnp.zeros((N_SLOT,), jnp.int32).at[sorted_src].set(
        jnp.arange(N_SLOT, dtype=jnp.int32))
    counts = jnp.bincount(slot_expert, length=E).astype(jnp.int32)
    meta = _tile_metadata(counts)

    xs = _sc_gather(xt, src_token)
    ro = _gmm(meta, xs, s_sorted,
              w1.astype(jnp.bfloat16), w3.astype(jnp.bfloat16),
              w2.astype(jnp.bfloat16))
    out = _sc_combine(ro, dest[:N_TOK], dest[N_TOK:])
    return out.reshape(bs, slen, dim)


# speedup vs baseline: 4.3317x; 1.0989x over previous
import functools

import jax
import jax.numpy as jnp
from jax import lax
from jax.experimental import pallas as pl
from jax.experimental.pallas import tpu as pltpu
from jax.experimental.pallas import tpu_sc as plsc

E = 8
K = 2
DIM = 1024
HID = 2816
N_TOK = 8192
N_SLOT = N_TOK * K

ROW_BLK = 256
NB = N_SLOT // ROW_BLK
T_MAX = NB + E - 1

NW = 32
GCH = 64
CCH = 32


def _router_body(x_ref, gw_ref, bias_ref, sel_ref, sc_ref):
    logits = jnp.dot(x_ref[...].astype(jnp.bfloat16),
                     gw_ref[...].astype(jnp.bfloat16),
                     preferred_element_type=jnp.float32)
    m = jnp.max(logits, axis=1, keepdims=True)
    ex = jnp.exp(logits - m)
    scores = ex / jnp.sum(ex, axis=1, keepdims=True)
    biased = scores + bias_ref[...]
    cols = lax.broadcasted_iota(jnp.int32, biased.shape, 1)
    i1 = jnp.argmax(biased, axis=1).astype(jnp.int32)
    m1 = cols == i1[:, None]
    b2 = jnp.where(m1, -jnp.inf, biased)
    i2 = jnp.argmax(b2, axis=1).astype(jnp.int32)
    s1 = jnp.sum(jnp.where(m1, scores, 0.0), axis=1)
    s2 = jnp.sum(jnp.where(cols == i2[:, None], scores, 0.0), axis=1)
    sel_ref[...] = jnp.stack([i1, i2])
    sc_ref[...] = jnp.stack([s1, s2])


def _router(xt, gw_t, bias2d):
    TB = 1024
    return pl.pallas_call(
        _router_body,
        grid=(N_TOK // TB,),
        in_specs=[
            pl.BlockSpec((TB, DIM), lambda i: (i, 0)),
            pl.BlockSpec((DIM, E), lambda i: (0, 0)),
            pl.BlockSpec((1, E), lambda i: (0, 0)),
        ],
        out_specs=[
            pl.BlockSpec((K, TB), lambda i: (0, i)),
            pl.BlockSpec((K, TB), lambda i: (0, i)),
        ],
        out_shape=[
            jax.ShapeDtypeStruct((K, N_TOK), jnp.int32),
            jax.ShapeDtypeStruct((K, N_TOK), jnp.float32),
        ],
    )(xt, gw_t, bias2d)


SPT = N_SLOT // NW
DCH = 64
NVR = SPT // 16


def _sc_dispatch_body(xt_hbm, sel_hbm, sco_hbm,
                      xs_hbm, s16_hbm, dest_hbm, counts_hbm,
                      sel_v, shadow_v, sco_v, hist_v, cnt16_v, histall_v,
                      dest2d_v, idx_bufs, xrows_v, s16_v, cnt_sm, shared, sem):
    c = lax.axis_index("c")
    s = lax.axis_index("s")
    wid = c * 16 + s
    base = wid * SPT
    shadow_base = ((1 - c) * 16 + s) * SPT
    iota16 = lax.iota(jnp.int32, 16)
    onehots = [1 - jnp.minimum(jnp.abs(iota16 - l), 1) for l in range(16)]

    pltpu.sync_copy(sel_hbm.at[pl.ds(base, SPT)], sel_v)
    pltpu.sync_copy(sel_hbm.at[pl.ds(shadow_base, SPT)], shadow_v)
    pltpu.sync_copy(sco_hbm.at[pl.ds(base, SPT)], sco_v)

    def hist_smem(buf):
        for e in range(16):
            cnt_sm[e] = 0

        def step(i, carry):
            v = buf[pl.ds(i * 16, 16)]
            for l in range(16):
                e = v[l]
                cnt_sm[e] = cnt_sm[e] + 1
            return carry
        lax.fori_loop(0, NVR, step, 0)
        h = jnp.zeros((16,), jnp.int32)
        for e in range(16):
            h = h + onehots[e] * cnt_sm[e]
        return h

    hist_v[pl.ds(0, 16)] = hist_smem(sel_v)
    pltpu.sync_copy(hist_v, shared.at[wid])
    hist_v[pl.ds(0, 16)] = hist_smem(shadow_v)
    pltpu.sync_copy(hist_v, shared.at[(1 - c) * 16 + s])
    plsc.subcore_barrier()

    pltpu.sync_copy(shared, histall_v)
    total = jnp.zeros((16,), jnp.int32)
    prefix = jnp.zeros((16,), jnp.int32)
    for w in range(NW):
        lt = jnp.minimum(jnp.maximum(wid - w, 0), 1)
        row = histall_v[w, pl.ds(0, 16)]
        total = total + row
        prefix = prefix + row * lt

    @pl.when(wid == 0)
    def _():
        cnt16_v[...] = total
        pltpu.sync_copy(cnt16_v, counts_hbm)

    acc = jnp.int32(0)
    for e in range(E):
        cnt_sm[e] = acc + prefix[e]
        acc = acc + total[e]

    for i in range(NVR):
        v = sel_v[pl.ds(i * 16, 16)]
        dest_v = jnp.zeros((16,), jnp.int32)
        for l in range(16):
            e = v[l]
            d = cnt_sm[e]
            cnt_sm[e] = d + 1
            dest_v = dest_v + onehots[l] * d
        dest2d_v[i // 4, pl.ds((i % 4) * 16, 16)] = dest_v
        idx_bufs[i // 4][pl.ds((i % 4) * 16, 16)] = dest_v
    pltpu.sync_copy(dest2d_v, dest_hbm.at[wid])

    tok_base = base - c * N_TOK
    for ci in range(SPT // DCH):
        pltpu.sync_copy(xt_hbm.at[pl.ds(tok_base + ci * DCH, DCH)], xrows_v)

        def srow16(j16, _):
            svec = sco_v[pl.ds(ci * DCH + j16 * 16, 16)]
            for l in range(16):
                s16_v[j16 * 16 + l, pl.ds(0, 16)] = jnp.full(
                    (16,), svec[l], jnp.float32)
            return 0
        lax.fori_loop(0, DCH // 16, srow16, 0)
        pltpu.async_copy(xrows_v, xs_hbm.at[idx_bufs[ci]], sem).wait()
        pltpu.async_copy(s16_v, s16_hbm.at[idx_bufs[ci]], sem).wait()


def _sc_dispatch(xt, sel_flat, sco_flat):
    return functools.partial(
        pl.kernel,
        out_type=(
            jax.ShapeDtypeStruct((N_SLOT, DIM), jnp.float32),
            jax.ShapeDtypeStruct((N_SLOT, 128), jnp.float32),
            jax.ShapeDtypeStruct((NW, SPT // DCH, DCH), jnp.int32),
            jax.ShapeDtypeStruct((16,), jnp.int32),
        ),
        mesh=plsc.VectorSubcoreMesh(core_axis_name="c", subcore_axis_name="s"),
        scratch_types=[
            pltpu.VMEM((SPT,), jnp.int32),
            pltpu.VMEM((SPT,), jnp.int32),
            pltpu.VMEM((SPT,), jnp.float32),
            pltpu.VMEM((128,), jnp.int32),
            pltpu.VMEM((16,), jnp.int32),
            pltpu.VMEM((32, 128), jnp.int32),
            pltpu.VMEM((SPT // DCH, DCH), jnp.int32),
            [pltpu.VMEM((DCH,), jnp.int32) for _ in range(SPT // DCH)],
            pltpu.VMEM((DCH, DIM), jnp.float32),
            pltpu.VMEM((DCH, 128), jnp.float32),
            pltpu.SMEM((16,), jnp.int32),
            pltpu.VMEM_SHARED((32, 128), jnp.int32),
            pltpu.SemaphoreType.DMA,
        ],
    )(_sc_dispatch_body)(xt, sel_flat, sco_flat)


def _gmm_body(meta_ref, xs_ref, s_ref, w1_ref, w3_ref, w2_ref, out_ref):
    t = pl.program_id(0)
    lo = meta_ref[2, t]
    hi = meta_ref[3, t]
    first = meta_ref[4, t]

    @pl.when(first == 1)
    def _():
        out_ref[...] = jnp.zeros_like(out_ref)

    @pl.when(hi > lo)
    def _():
        xb = (xs_ref[...] * s_ref[:, 0:1]).astype(jnp.bfloat16)
        a = jnp.dot(xb, w1_ref[0], preferred_element_type=jnp.float32)
        b = jnp.dot(xb, w3_ref[0], preferred_element_type=jnp.float32)
        h = (a * jax.nn.sigmoid(a) * b).astype(jnp.bfloat16)
        o = jnp.dot(h, w2_ref[0], preferred_element_type=jnp.float32)
        rows = lax.broadcasted_iota(jnp.int32, (ROW_BLK, DIM), 0)
        mask = (rows >= lo) & (rows < hi)
        out_ref[...] = out_ref[...] + jnp.where(mask, o, 0.0)


def _gmm(meta, xs, s_sorted, w1b, w3b, w2b):
    grid_spec = pltpu.PrefetchScalarGridSpec(
        num_scalar_prefetch=1,
        grid=(T_MAX,),
        in_specs=[
            pl.BlockSpec((ROW_BLK, DIM), lambda t, m: (m[1, t], 0)),
            pl.BlockSpec((ROW_BLK, 128), lambda t, m: (m[1, t], 0)),
            pl.BlockSpec((1, DIM, HID), lambda t, m: (m[0, t], 0, 0)),
            pl.BlockSpec((1, DIM, HID), lambda t, m: (m[0, t], 0, 0)),
            pl.BlockSpec((1, HID, DIM), lambda t, m: (m[0, t], 0, 0)),
        ],
        out_specs=pl.BlockSpec((ROW_BLK, DIM), lambda t, m: (m[1, t], 0)),
    )
    return pl.pallas_call(
        _gmm_body,
        grid_spec=grid_spec,
        out_shape=jax.ShapeDtypeStruct((N_SLOT, DIM), jnp.float32),
    )(meta, xs, s_sorted, w1b, w3b, w2b)


def _sc_combine_body(ro_hbm, d0_hbm, d1_hbm, out_hbm, i0_v, i1_v, r0_v, r1_v, sem):
    wid = lax.axis_index("s") * 2 + lax.axis_index("c")
    base = wid * (N_TOK // NW)
    for c in range(N_TOK // NW // CCH):
        o = base + c * CCH
        pltpu.sync_copy(d0_hbm.at[pl.ds(o, CCH)], i0_v)
        pltpu.sync_copy(d1_hbm.at[pl.ds(o, CCH)], i1_v)
        pltpu.async_copy(ro_hbm.at[i0_v], r0_v, sem).wait()
        pltpu.async_copy(ro_hbm.at[i1_v], r1_v, sem).wait()

        def row_add(i, _):
            for j in range(DIM // 16):
                sl = pl.ds(j * 16, 16)
                r0_v[i, sl] = r0_v[i, sl] + r1_v[i, sl]
            return 0

        lax.fori_loop(0, CCH, row_add, 0)
        pltpu.sync_copy(r0_v, out_hbm.at[pl.ds(o, CCH)])


def _sc_combine(ro, d0, d1):
    return functools.partial(
        pl.kernel,
        out_type=jax.ShapeDtypeStruct((N_TOK, DIM), jnp.float32),
        mesh=plsc.VectorSubcoreMesh(core_axis_name="c", subcore_axis_name="s"),
        scratch_types=[
            pltpu.VMEM((CCH,), jnp.int32),
            pltpu.VMEM((CCH,), jnp.int32),
            pltpu.VMEM((CCH, DIM), jnp.float32),
            pltpu.VMEM((CCH, DIM), jnp.float32),
            pltpu.SemaphoreType.DMA,
        ],
    )(_sc_combine_body)(ro, d0, d1)


def _tile_metadata(counts):
    counts = counts.astype(jnp.int32)
    off = jnp.concatenate([jnp.zeros((1,), jnp.int32), jnp.cumsum(counts)])
    start_blk = off[:E] // ROW_BLK
    end_blk = (off[1:] + ROW_BLK - 1) // ROW_BLK
    nblk = jnp.where(counts > 0, end_blk - start_blk, 0)
    ts = jnp.concatenate([jnp.zeros((1,), jnp.int32),
                          jnp.cumsum(nblk)[:-1].astype(jnp.int32)])
    total = ts[E - 1] + nblk[E - 1]
    t = jnp.arange(T_MAX, dtype=jnp.int32)
    e_t = jnp.clip(jnp.searchsorted(ts, t, side="right").astype(jnp.int32) - 1, 0, E - 1)
    blk = start_blk[e_t] + (t - ts[e_t])
    active = t < total
    lo = jnp.maximum(off[e_t], blk * ROW_BLK) - blk * ROW_BLK
    hi = jnp.minimum(off[e_t + 1], (blk + 1) * ROW_BLK) - blk * ROW_BLK
    lo = jnp.where(active, lo, 0)
    hi = jnp.where(active, hi, 0)
    first = (active & (off[e_t] <= blk * ROW_BLK)).astype(jnp.int32)
    blk = jnp.where(active, blk, NB - 1)
    return jnp.stack([e_t, blk, lo, hi, first]).astype(jnp.int32)


def kernel(x, gate_w, w1, w2, w3, expert_bias):
    bs, slen, dim = x.shape
    xt = x.reshape(N_TOK, DIM)
    sel, scores = _router(xt, gate_w.T, expert_bias.reshape(1, E))

    xs, s16, dest3d, counts16 = _sc_dispatch(
        xt, sel.reshape(N_SLOT), scores.reshape(N_SLOT))
    dest = dest3d.reshape(N_SLOT)
    meta = _tile_metadata(counts16[:E])

    ro = _gmm(meta, xs, s16,
              w1.astype(jnp.bfloat16), w3.astype(jnp.bfloat16),
              w2.astype(jnp.bfloat16))
    out = _sc_combine(ro, dest[:N_TOK], dest[N_TOK:])
    return out.reshape(bs, slen, dim)
